# baseline (speedup 1.0000x reference)
import functools

import jax
import jax.numpy as jnp
from jax.experimental import pallas as pl
from jax.experimental.pallas import tpu as pltpu
from jax.experimental.pallas import tpu_sc as plsc

_EMBED = 64
_NMETA = 16
_CTX = 128
_WIN = 128
_MBLK = 64


def _sc_gather(emb_table, idx2d):
    nwin = idx2d.shape[0]
    n = nwin * _WIN
    mesh = plsc.VectorSubcoreMesh(core_axis_name="c", subcore_axis_name="s")

    @functools.partial(
        pl.kernel,
        out_type=jax.ShapeDtypeStruct((n, _CTX), emb_table.dtype),
        mesh=mesh,
        compiler_params=pltpu.CompilerParams(use_tc_tiling_on_sc=False),
    )
    def gather_kernel(tbl_hbm, idx_hbm, out_hbm):
        def body(i_vmem, o_vmem):
            pltpu.sync_copy(tbl_hbm.at[i_vmem.at[0]], o_vmem)

        pltpu.emit_pipeline(
            body,
            grid=(nwin,),
            in_specs=[
                pl.BlockSpec((1, _WIN), index_map=lambda i: (i, 0))
            ],
            out_specs=[
                pl.BlockSpec((_WIN, _EMBED), index_map=lambda i: (i, 0))
            ],
            core_axis_name=("c", "s"),
            dimension_semantics=(pltpu.PARALLEL,),
        )(idx_hbm, out_hbm)

    return gather_kernel(emb_table, idx2d)


def _tc_body(emb_ref, m8_ref, w1al_ref, w1b8_ref, b1_ref, w2_ref, b2_ref,
             gamma_ref, beta_ref, out_ref):
    p, ns, nm = emb_ref.shape[0], emb_ref.shape[1], emb_ref.shape[2]
    bf = jnp.bfloat16
    rows = p * nm
    lane = jax.lax.broadcasted_iota(jnp.int32, (rows, _CTX), 1)
    m8 = m8_ref[...].reshape(rows, _CTX).astype(bf)
    w1al = w1al_ref[...].astype(bf)
    w2 = w2_ref[...].astype(bf)
    for j in range(ns):
        x = emb_ref[:, j].reshape(rows, _CTX)
        x = jnp.where(lane < _EMBED, x, 0.0).astype(bf)
        h = (
            jnp.dot(x, w1al, preferred_element_type=jnp.float32)
            + jnp.dot(m8, w1b8_ref[j].astype(bf),
                      preferred_element_type=jnp.float32)
            + b1_ref[...]
        )
        h = jnp.maximum(h, 0.0)
        h = jnp.dot(h.astype(bf), w2,
                    preferred_element_type=jnp.float32) + b2_ref[...]
        mean = jnp.mean(h, axis=-1, keepdims=True)
        d = h - mean
        var = jnp.mean(d * d, axis=-1, keepdims=True)
        y = d * jax.lax.rsqrt(var + 1e-5) * gamma_ref[...] + beta_ref[...]
        out_ref[j] = jnp.max(y.reshape(p, nm, _CTX), axis=0)


def _tc_mlp(emb4, m83, w1al, w1b8, b1, w2, b2, gamma, beta):
    p, ns, nm = emb4.shape[0], emb4.shape[1], emb4.shape[2]
    fixed = lambda i: (0, 0)
    return pl.pallas_call(
        _tc_body,
        grid=(nm // _MBLK,),
        in_specs=[
            pl.BlockSpec((p, ns, _MBLK, _CTX), lambda i: (0, 0, i, 0)),
            pl.BlockSpec((p, _MBLK, _CTX), lambda i: (0, i, 0)),
            pl.BlockSpec((_CTX, _CTX), fixed),
            pl.BlockSpec((ns, _CTX, _CTX), lambda i: (0, 0, 0)),
            pl.BlockSpec((1, _CTX), fixed),
            pl.BlockSpec((_CTX, _CTX), fixed),
            pl.BlockSpec((1, _CTX), fixed),
            pl.BlockSpec((1, _CTX), fixed),
            pl.BlockSpec((1, _CTX), fixed),
        ],
        out_specs=pl.BlockSpec((ns, _MBLK, _CTX), lambda i: (0, i, 0)),
        out_shape=jax.ShapeDtypeStruct((ns, nm, _CTX), jnp.float32),
    )(emb4, m83, w1al, w1b8, b1, w2, b2, gamma, beta)


_NCHUNK = 2


def kernel(pattern_ids, pattern_metadata, emb_table, W1, b1, W2, b2, gamma, beta):
    bsz, p = pattern_ids.shape
    nm = bsz // 8
    ids_g = pattern_ids.T.reshape(p, nm, 8).transpose(0, 2, 1)
    m83 = pattern_metadata.transpose(1, 0, 2).reshape(p, nm, 8 * _NMETA)
    w1a = W1[:_EMBED]
    w1b = W1[_EMBED:]
    w1al = jnp.concatenate([w1a, jnp.zeros((_CTX - _EMBED, _CTX), W1.dtype)], axis=0)
    w1b8 = jnp.zeros((8, _CTX, _CTX), W1.dtype)
    for j in range(8):
        w1b8 = w1b8.at[j, j * _NMETA:(j + 1) * _NMETA, :].set(w1b)
    b1r, b2r = b1.reshape(1, _CTX), b2.reshape(1, _CTX)
    gr, br = gamma.reshape(1, _CTX), beta.reshape(1, _CTX)
    mc = nm // _NCHUNK
    pools = []
    for c in range(_NCHUNK):
        ids_c = ids_g[:, :, c * mc:(c + 1) * mc]
        idx2d = ids_c.reshape(p * 8 * mc // _WIN, _WIN).astype(jnp.int32)
        embeds = _sc_gather(emb_table, idx2d)
        emb4 = embeds.reshape(p, 8, mc, _CTX)
        pools.append(_tc_mlp(
            emb4, m83[:, c * mc:(c + 1) * mc, :], w1al, w1b8,
            b1r, W2, b2r, gr, br,
        ))
    pooled = jnp.concatenate(pools, axis=1)
    return pooled.transpose(1, 0, 2).reshape(bsz, _CTX)


# speedup vs baseline: 1.0270x; 1.0270x over previous
import functools

import jax
import jax.numpy as jnp
from jax.experimental import pallas as pl
from jax.experimental.pallas import tpu as pltpu
from jax.experimental.pallas import tpu_sc as plsc

_EMBED = 64
_NMETA = 16
_CTX = 128
_WIN = 128
_MBLK = 64


def _sc_gather(emb_table, idx2d):
    nwin = idx2d.shape[0]
    n = nwin * _WIN
    mesh = plsc.VectorSubcoreMesh(core_axis_name="c", subcore_axis_name="s")

    @functools.partial(
        pl.kernel,
        out_type=jax.ShapeDtypeStruct((n, _CTX), emb_table.dtype),
        mesh=mesh,
        compiler_params=pltpu.CompilerParams(use_tc_tiling_on_sc=False),
    )
    def gather_kernel(tbl_hbm, idx_hbm, out_hbm):
        def body(i_vmem, o_vmem):
            pltpu.sync_copy(tbl_hbm.at[i_vmem.at[0]], o_vmem)

        pltpu.emit_pipeline(
            body,
            grid=(nwin,),
            in_specs=[
                pl.BlockSpec((1, _WIN), index_map=lambda i: (i, 0))
            ],
            out_specs=[
                pl.BlockSpec((_WIN, _EMBED), index_map=lambda i: (i, 0))
            ],
            core_axis_name=("c", "s"),
            dimension_semantics=(pltpu.PARALLEL,),
        )(idx_hbm, out_hbm)

    return gather_kernel(emb_table, idx2d)


def _tc_body(emb_ref, m8_ref, w1al_ref, w1b8_ref, b1_ref, w2_ref, b2_ref,
             gamma_ref, beta_ref, out_ref):
    p, ns, nm = emb_ref.shape[0], emb_ref.shape[1], emb_ref.shape[2]
    bf = jnp.bfloat16
    rows = p * nm
    lane = jax.lax.broadcasted_iota(jnp.int32, (rows, _CTX), 1)
    m8 = m8_ref[...].reshape(rows, _CTX).astype(bf)
    w1al = w1al_ref[...].astype(bf)
    w2 = w2_ref[...].astype(bf)
    for j in range(ns):
        x = emb_ref[:, j].reshape(rows, _CTX)
        x = jnp.where(lane < _EMBED, x, 0.0).astype(bf)
        h = (
            jnp.dot(x, w1al, preferred_element_type=jnp.float32)
            + jnp.dot(m8, w1b8_ref[j].astype(bf),
                      preferred_element_type=jnp.float32)
            + b1_ref[...]
        )
        h = jnp.maximum(h, 0.0)
        h = jnp.dot(h.astype(bf), w2,
                    preferred_element_type=jnp.float32) + b2_ref[...]
        mean = jnp.mean(h, axis=-1, keepdims=True)
        d = h - mean
        var = jnp.mean(d * d, axis=-1, keepdims=True)
        y = d * jax.lax.rsqrt(var + 1e-5) * gamma_ref[...] + beta_ref[...]
        out_ref[j] = jnp.max(y.reshape(p, nm, _CTX), axis=0)


def _tc_mlp(emb4, m83, w1al, w1b8, b1, w2, b2, gamma, beta):
    p, ns, nm = emb4.shape[0], emb4.shape[1], emb4.shape[2]
    fixed = lambda i: (0, 0)
    return pl.pallas_call(
        _tc_body,
        grid=(nm // _MBLK,),
        in_specs=[
            pl.BlockSpec((p, ns, _MBLK, _CTX), lambda i: (0, 0, i, 0)),
            pl.BlockSpec((p, _MBLK, _CTX), lambda i: (0, i, 0)),
            pl.BlockSpec((_CTX, _CTX), fixed),
            pl.BlockSpec((ns, _CTX, _CTX), lambda i: (0, 0, 0)),
            pl.BlockSpec((1, _CTX), fixed),
            pl.BlockSpec((_CTX, _CTX), fixed),
            pl.BlockSpec((1, _CTX), fixed),
            pl.BlockSpec((1, _CTX), fixed),
            pl.BlockSpec((1, _CTX), fixed),
        ],
        out_specs=pl.BlockSpec((ns, _MBLK, _CTX), lambda i: (0, i, 0)),
        out_shape=jax.ShapeDtypeStruct((ns, nm, _CTX), jnp.float32),
    )(emb4, m83, w1al, w1b8, b1, w2, b2, gamma, beta)


def kernel(pattern_ids, pattern_metadata, emb_table, W1, b1, W2, b2, gamma, beta):
    bsz, p = pattern_ids.shape
    n = bsz * p
    nm = bsz // 8
    ids_g = pattern_ids.T.reshape(p, nm, 8).transpose(0, 2, 1)
    idx2d = ids_g.reshape(n // _WIN, _WIN).astype(jnp.int32)
    embeds = _sc_gather(emb_table, idx2d)
    emb4 = embeds.reshape(p, 8, nm, _CTX)
    m83 = pattern_metadata.transpose(1, 0, 2).reshape(p, nm, 8 * _NMETA)
    w1a = W1[:_EMBED]
    w1b = W1[_EMBED:]
    w1al = jnp.concatenate([w1a, jnp.zeros((_CTX - _EMBED, _CTX), W1.dtype)], axis=0)
    w1b8 = jnp.zeros((8, _CTX, _CTX), W1.dtype)
    for j in range(8):
        w1b8 = w1b8.at[j, j * _NMETA:(j + 1) * _NMETA, :].set(w1b)
    pooled = _tc_mlp(
        emb4, m83, w1al, w1b8,
        b1.reshape(1, _CTX), W2, b2.reshape(1, _CTX),
        gamma.reshape(1, _CTX), beta.reshape(1, _CTX),
    )
    return pooled.transpose(1, 0, 2).reshape(bsz, _CTX)


# speedup vs baseline: 1.0367x; 1.0094x over previous
import functools

import jax
import jax.numpy as jnp
from jax.experimental import pallas as pl
from jax.experimental.pallas import tpu as pltpu
from jax.experimental.pallas import tpu_sc as plsc

_EMBED = 64
_NMETA = 16
_CTX = 128
_WIN = 128
_MBLK = 64


def _sc_gather(emb_table, idx2d):
    nwin = idx2d.shape[0]
    n = nwin * _WIN
    mesh = plsc.VectorSubcoreMesh(core_axis_name="c", subcore_axis_name="s")

    @functools.partial(
        pl.kernel,
        out_type=jax.ShapeDtypeStruct((n, _CTX), emb_table.dtype),
        mesh=mesh,
        compiler_params=pltpu.CompilerParams(use_tc_tiling_on_sc=False),
    )
    def gather_kernel(tbl_hbm, idx_hbm, out_hbm):
        def body(i_vmem, o_vmem):
            pltpu.sync_copy(tbl_hbm.at[i_vmem.at[0]],
                            o_vmem.at[pl.ds(0, _WIN)])
            pltpu.sync_copy(tbl_hbm.at[i_vmem.at[1]],
                            o_vmem.at[pl.ds(_WIN, _WIN)])

        pltpu.emit_pipeline(
            body,
            grid=(nwin // 2,),
            in_specs=[
                pl.BlockSpec((2, _WIN), index_map=lambda i: (i, 0))
            ],
            out_specs=[
                pl.BlockSpec((2 * _WIN, _EMBED), index_map=lambda i: (i, 0))
            ],
            core_axis_name=("c", "s"),
            dimension_semantics=(pltpu.PARALLEL,),
        )(idx_hbm, out_hbm)

    return gather_kernel(emb_table, idx2d)


def _tc_body(emb_ref, m8_ref, w1al_ref, w1b8_ref, b1_ref, w2_ref, b2_ref,
             gamma_ref, beta_ref, out_ref):
    p, ns, nm = emb_ref.shape[0], emb_ref.shape[1], emb_ref.shape[2]
    bf = jnp.bfloat16
    rows = p * nm
    lane = jax.lax.broadcasted_iota(jnp.int32, (rows, _CTX), 1)
    m8 = m8_ref[...].reshape(rows, _CTX).astype(bf)
    w1al = w1al_ref[...].astype(bf)
    w2 = w2_ref[...].astype(bf)
    for j in range(ns):
        x = emb_ref[:, j].reshape(rows, _CTX)
        x = jnp.where(lane < _EMBED, x, 0.0).astype(bf)
        h = (
            jnp.dot(x, w1al, preferred_element_type=jnp.float32)
            + jnp.dot(m8, w1b8_ref[j].astype(bf),
                      preferred_element_type=jnp.float32)
            + b1_ref[...]
        )
        h = jnp.maximum(h, 0.0)
        h = jnp.dot(h.astype(bf), w2,
                    preferred_element_type=jnp.float32) + b2_ref[...]
        mean = jnp.mean(h, axis=-1, keepdims=True)
        d = h - mean
        var = jnp.mean(d * d, axis=-1, keepdims=True)
        y = d * jax.lax.rsqrt(var + 1e-5) * gamma_ref[...] + beta_ref[...]
        out_ref[j] = jnp.max(y.reshape(p, nm, _CTX), axis=0)


def _tc_mlp(emb4, m83, w1al, w1b8, b1, w2, b2, gamma, beta):
    p, ns, nm = emb4.shape[0], emb4.shape[1], emb4.shape[2]
    fixed = lambda i: (0, 0)
    return pl.pallas_call(
        _tc_body,
        grid=(nm // _MBLK,),
        in_specs=[
            pl.BlockSpec((p, ns, _MBLK, _CTX), lambda i: (0, 0, i, 0)),
            pl.BlockSpec((p, _MBLK, _CTX), lambda i: (0, i, 0)),
            pl.BlockSpec((_CTX, _CTX), fixed),
            pl.BlockSpec((ns, _CTX, _CTX), lambda i: (0, 0, 0)),
            pl.BlockSpec((1, _CTX), fixed),
            pl.BlockSpec((_CTX, _CTX), fixed),
            pl.BlockSpec((1, _CTX), fixed),
            pl.BlockSpec((1, _CTX), fixed),
            pl.BlockSpec((1, _CTX), fixed),
        ],
        out_specs=pl.BlockSpec((ns, _MBLK, _CTX), lambda i: (0, i, 0)),
        out_shape=jax.ShapeDtypeStruct((ns, nm, _CTX), jnp.float32),
    )(emb4, m83, w1al, w1b8, b1, w2, b2, gamma, beta)


def kernel(pattern_ids, pattern_metadata, emb_table, W1, b1, W2, b2, gamma, beta):
    bsz, p = pattern_ids.shape
    n = bsz * p
    nm = bsz // 8
    ids_g = pattern_ids.T.reshape(p, nm, 8).transpose(0, 2, 1)
    idx2d = ids_g.reshape(n // _WIN, _WIN).astype(jnp.int32)
    embeds = _sc_gather(emb_table, idx2d)
    emb4 = embeds.reshape(p, 8, nm, _CTX)
    m83 = pattern_metadata.transpose(1, 0, 2).reshape(p, nm, 8 * _NMETA)
    w1a = W1[:_EMBED]
    w1b = W1[_EMBED:]
    w1al = jnp.concatenate([w1a, jnp.zeros((_CTX - _EMBED, _CTX), W1.dtype)], axis=0)
    w1b8 = jnp.zeros((8, _CTX, _CTX), W1.dtype)
    for j in range(8):
        w1b8 = w1b8.at[j, j * _NMETA:(j + 1) * _NMETA, :].set(w1b)
    pooled = _tc_mlp(
        emb4, m83, w1al, w1b8,
        b1.reshape(1, _CTX), W2, b2.reshape(1, _CTX),
        gamma.reshape(1, _CTX), beta.reshape(1, _CTX),
    )
    return pooled.transpose(1, 0, 2).reshape(bsz, _CTX)
